# baseline (speedup 1.0000x reference)
import jax
import jax.numpy as jnp
import numpy as np
from jax import lax
from jax.experimental import pallas as pl
from jax.experimental.pallas import tpu as pltpu

B, S, D, H = 4, 4096, 1024, 16
DH = D // H
BUCKET = 64
NH = 4
NCH = S // BUCKET
DFF = 4096
NINST = B * H * NH
SCALE = 1.0 / np.sqrt(DH)

BM = 512


def _k1_body(src_ref, wqk_ref, wv_ref, rot_ref, qk_ref, v_ref, bkt_ref):
    x = src_ref[0]
    qk = jnp.dot(x, wqk_ref[...], preferred_element_type=jnp.float32)
    v = jnp.dot(x, wv_ref[...], preferred_element_type=jnp.float32)
    qk_ref[0] = qk.astype(jnp.bfloat16)
    v_ref[0] = v.astype(jnp.bfloat16)
    rot2 = rot_ref[...]
    iota = lax.broadcasted_iota(jnp.int32, (BM, 2 * 32), 1)
    parts = []
    for h in range(H):
        qh = qk[:, h * DH:(h + 1) * DH]
        n = jnp.sqrt(jnp.sum(qh * qh, axis=1, keepdims=True))
        qn = qh / (n + 1e-6)
        rh = jnp.dot(qn, rot2, preferred_element_type=jnp.float32)
        for r in range(NH):
            vals = rh[:, r * 32:(r + 1) * 32]
            cat = jnp.concatenate([vals, -vals], axis=1)
            m = jnp.max(cat, axis=1, keepdims=True)
            idx = jnp.min(jnp.where(cat >= m, iota, 2 * 32), axis=1,
                          keepdims=True)
            parts.append(idx)
    bkt_ref[0] = jnp.concatenate(parts, axis=1)


def k1_proj_hash(src, Wqk, Wv, rot):
    rot2 = rot.reshape(DH, NH * 32)
    return pl.pallas_call(
        _k1_body,
        grid=(B, S // BM),
        in_specs=[
            pl.BlockSpec((1, BM, D), lambda b, s: (b, s, 0)),
            pl.BlockSpec((D, D), lambda b, s: (0, 0)),
            pl.BlockSpec((D, D), lambda b, s: (0, 0)),
            pl.BlockSpec((DH, NH * 32), lambda b, s: (0, 0)),
        ],
        out_specs=[
            pl.BlockSpec((1, BM, D), lambda b, s: (b, s, 0)),
            pl.BlockSpec((1, BM, D), lambda b, s: (b, s, 0)),
            pl.BlockSpec((1, BM, H * NH), lambda b, s: (b, s, 0)),
        ],
        out_shape=[
            jax.ShapeDtypeStruct((B, S, D), jnp.bfloat16),
            jax.ShapeDtypeStruct((B, S, D), jnp.bfloat16),
            jax.ShapeDtypeStruct((B, S, H * NH), jnp.int32),
        ],
    )(src, Wqk, Wv, rot2)


TS = 128
NT = S // TS


def _k2_body(bkt_ref, inv_ref):
    bkt = bkt_ref[0]
    totals = []
    for beta in range(BUCKET):
        totals.append(jnp.sum((bkt == beta).astype(jnp.float32), axis=0,
                              keepdims=True))
    offs = []
    run = jnp.zeros((1, H * NH), jnp.float32)
    for beta in range(BUCKET):
        offs.append(run)
        run = run + totals[beta]
    ii = lax.broadcasted_iota(jnp.int32, (TS, TS), 0)
    jj = lax.broadcasted_iota(jnp.int32, (TS, TS), 1)
    T = (jj <= ii).astype(jnp.float32)

    def tile_step(t, carrys):
        blk = bkt_ref[0, pl.ds(t * TS, TS), :]
        acc = jnp.zeros((TS, H * NH), jnp.float32)
        new_carrys = []
        for beta in range(BUCKET):
            I = (blk == beta).astype(jnp.float32)
            incl = jnp.dot(T, I, preferred_element_type=jnp.float32)
            acc = acc + I * (incl - 1.0 + carrys[beta] + offs[beta])
            new_carrys.append(carrys[beta] + incl[TS - 1:TS, :])
        inv_ref[0, pl.ds(t * TS, TS), :] = acc.astype(jnp.int32)
        return tuple(new_carrys)

    lax.fori_loop(0, NT, tile_step,
                  tuple(jnp.zeros((1, H * NH), jnp.float32)
                        for _ in range(BUCKET)))


def k2_inv(bkt):
    return pl.pallas_call(
        _k2_body,
        grid=(B,),
        in_specs=[pl.BlockSpec((1, S, H * NH), lambda b: (b, 0, 0))],
        out_specs=pl.BlockSpec((1, S, H * NH), lambda b: (b, 0, 0)),
        out_shape=jax.ShapeDtypeStruct((B, S, H * NH), jnp.int32),
    )(bkt)


CG = 4
QB = CG * BUCKET
KB = QB + BUCKET
NB4 = NCH // CG


def _norm_bf16(x_bf):
    x = x_bf.astype(jnp.float32)
    n = jnp.sqrt(jnp.sum(x * x, axis=1, keepdims=True))
    return (x * (1.0 / (n + 1e-6))).astype(jnp.bfloat16)


def _k4_body(qk_ref, qkh_ref, v_ref, vh_ref, o_ref, lse_ref):
    ri = lax.broadcasted_iota(jnp.int32, (QB, KB), 0)
    ci = lax.broadcasted_iota(jnp.int32, (QB, KB), 1)
    band = (ci // BUCKET == ri // BUCKET) | (ci // BUCKET == ri // BUCKET + 1)
    dead = (~band) | (ci == ri + BUCKET)
    maskmat = jnp.where(dead, -1e5, 0.0).astype(jnp.float32)

    qc = qk_ref[0]
    kcat = jnp.concatenate([_norm_bf16(qkh_ref[0]), _norm_bf16(qc)], axis=0)
    vcat = jnp.concatenate([vh_ref[0], v_ref[0]], axis=0)
    dots = lax.dot_general(qc, kcat, (((1,), (1,)), ((), ())),
                           preferred_element_type=jnp.float32)
    dots = dots * SCALE + maskmat
    p_ = jnp.exp(dots)
    s = jnp.sum(p_, axis=1, keepdims=True)
    o = jnp.dot(p_.astype(jnp.bfloat16), vcat,
                preferred_element_type=jnp.float32) * (1.0 / s)
    o_ref[0] = o.astype(jnp.bfloat16)
    lse_ref[0] = jnp.broadcast_to(jnp.log(s), (QB, 16))


def k4_attention(qk_s, v_s):
    def main(i, cb):
        return (i, cb, 0)

    def halo(i, cb):
        return (i, (cb * CG * BUCKET - BUCKET) % S // BUCKET, 0)

    return pl.pallas_call(
        _k4_body,
        grid=(NINST, NB4),
        in_specs=[
            pl.BlockSpec((1, QB, DH), main),
            pl.BlockSpec((1, BUCKET, DH), halo),
            pl.BlockSpec((1, QB, DH), main),
            pl.BlockSpec((1, BUCKET, DH), halo),
        ],
        out_specs=[
            pl.BlockSpec((1, QB, DH), main),
            pl.BlockSpec((1, QB, 16), main),
        ],
        out_shape=[
            jax.ShapeDtypeStruct((NINST, S, DH), jnp.bfloat16),
            jax.ShapeDtypeStruct((NINST, S, 16), jnp.float32),
        ],
    )(qk_s, qk_s, v_s, v_s)


BMC = 128


def _k6_body(o_ref, lse_ref, wo_ref, out_ref):
    ls = [lse_ref[r, 0] for r in range(NH)]
    m = ls[0]
    for r in range(1, NH):
        m = jnp.maximum(m, ls[r])
    es = [jnp.exp(l - m) for l in ls]
    ssum = es[0]
    for r in range(1, NH):
        ssum = ssum + es[r]
    bmat = jnp.full((16, DH), 1.0 / 16.0, jnp.float32)
    parts = []
    for h in range(H):
        acc = jnp.zeros((BMC, DH), jnp.float32)
        for r in range(NH):
            w = jnp.dot(es[r][h] * (1.0 / ssum[h]), bmat,
                        preferred_element_type=jnp.float32)
            acc = acc + o_ref[r, 0, :, h * DH:(h + 1) * DH].astype(
                jnp.float32) * w
        parts.append(acc)
    attn = jnp.concatenate(parts, axis=1)
    out_ref[0] = jnp.dot(attn, wo_ref[...], preferred_element_type=jnp.float32)


def k6_combine_wo(o_u, lse_u, Wo):
    return pl.pallas_call(
        _k6_body,
        grid=(B, S // BMC),
        in_specs=[
            pl.BlockSpec((NH, 1, BMC, D), lambda b, s: (0, b, s, 0)),
            pl.BlockSpec((NH, 1, H, BMC, 16), lambda b, s: (0, b, 0, s, 0)),
            pl.BlockSpec((D, D), lambda b, s: (0, 0)),
        ],
        out_specs=pl.BlockSpec((1, BMC, D), lambda b, s: (b, s, 0)),
        out_shape=jax.ShapeDtypeStruct((B, S, D), jnp.float32),
    )(o_u, lse_u, Wo)


FT = 1024


def _k7_body(x_ref, w1_ref, b1_ref, w2_ref, b2_ref, o_ref):
    t = pl.program_id(2)
    h = jnp.dot(x_ref[0], w1_ref[...], preferred_element_type=jnp.float32)
    h = jnp.maximum(h + b1_ref[...], 0.0)
    part = jnp.dot(h, w2_ref[...], preferred_element_type=jnp.float32)

    @pl.when(t == 0)
    def _():
        o_ref[0] = part + b2_ref[...]

    @pl.when(t != 0)
    def _():
        o_ref[0] = o_ref[0] + part


def k7_ffn(x, w1, b1, w2, b2):
    return pl.pallas_call(
        _k7_body,
        grid=(B, S // BM, DFF // FT),
        in_specs=[
            pl.BlockSpec((1, BM, D), lambda b, s, t: (b, s, 0)),
            pl.BlockSpec((D, FT), lambda b, s, t: (0, t)),
            pl.BlockSpec((1, FT), lambda b, s, t: (0, t)),
            pl.BlockSpec((FT, D), lambda b, s, t: (t, 0)),
            pl.BlockSpec((1, D), lambda b, s, t: (0, 0)),
        ],
        out_specs=pl.BlockSpec((1, BM, D), lambda b, s, t: (b, s, 0)),
        out_shape=jax.ShapeDtypeStruct((B, S, D), jnp.float32),
    )(x, w1, b1.reshape(1, DFF), w2, b2.reshape(1, D))

from jax.experimental.pallas import tpu_sc as plsc
import functools

NW = 32
IPW = NINST // NW
SR = S // 128
QR = 4
NQ = SR // QR

def _sc_mesh():
    return plsc.VectorSubcoreMesh(core_axis_name="c", subcore_axis_name="s")


def _k3_body(inv_hbm, qk4, v4, qk_s4, v_s4, inv_v, src_v, qbuf, vbuf, sem):
    wid = lax.axis_index("s") * 2 + lax.axis_index("c")

    def inst_body(k, carry):
        inst = wid * IPW + k
        b = inst // (H * NH)
        c = inst % (H * NH)
        h = c // NH
        pltpu.sync_copy(inv_hbm.at[inst], inv_v)

        def mkidx(rr, carry2):
            for j in range(8):
                i0 = rr * 128 + j * 16
                src_v[rr, pl.ds(j * 16, 16)] = (
                    (lax.iota(jnp.int32, 16) + i0) * H + h)
            return carry2

        lax.fori_loop(0, SR, mkidx, 0)
        for q in range(NQ):
            cps = []
            for j in range(QR):
                g = q * QR + j
                cps.append(pltpu.async_copy(
                    qk4.at[b].at[src_v.at[g]],
                    qbuf.at[pl.ds(j * 128, 128)], sem))
                cps.append(pltpu.async_copy(
                    v4.at[b].at[src_v.at[g]],
                    vbuf.at[pl.ds(j * 128, 128)], sem))
            for cp in cps:
                cp.wait()
            cps = []
            for j in range(QR):
                g = q * QR + j
                cps.append(pltpu.async_copy(
                    qbuf.at[pl.ds(j * 128, 128)],
                    qk_s4.at[inst].at[inv_v.at[g]], sem))
                cps.append(pltpu.async_copy(
                    vbuf.at[pl.ds(j * 128, 128)],
                    v_s4.at[inst].at[inv_v.at[g]], sem))
            for cp in cps:
                cp.wait()
        return carry

    lax.fori_loop(0, IPW, inst_body, 0)


def k3_sort_gather(inv2, qk, v):
    qk4 = qk.reshape(B, S * H, DH)
    v4 = v.reshape(B, S * H, DH)
    f = pl.kernel(
        _k3_body,
        mesh=_sc_mesh(),
        compiler_params=pltpu.CompilerParams(use_tc_tiling_on_sc=False),
        out_type=[
            jax.ShapeDtypeStruct((NINST, S, DH), jnp.bfloat16),
            jax.ShapeDtypeStruct((NINST, S, DH), jnp.bfloat16),
        ],
        scratch_types=[
            pltpu.VMEM((SR, 128), jnp.int32),
            pltpu.VMEM((SR, 128), jnp.int32),
            pltpu.VMEM((QR * 128, DH), jnp.bfloat16),
            pltpu.VMEM((QR * 128, DH), jnp.bfloat16),
            pltpu.SemaphoreType.DMA,
        ],
    )
    return f(inv2, qk4, v4)


def _k5_body(inv_hbm, o_s3, lse_s3, o_u3, lse_u2,
             inv_v, dst_v, obuf, lbuf, sem):
    wid = lax.axis_index("s") * 2 + lax.axis_index("c")

    def inst_body(k, carry):
        inst = wid * IPW + k
        b = inst // (H * NH)
        c = inst % (H * NH)
        h = c // NH
        r = c % NH
        rb = r * B + b
        base_l = ((rb * H) + h) * S
        pltpu.sync_copy(inv_hbm.at[inst], inv_v)

        def mkidx(rr, carry2):
            for j in range(8):
                i0 = rr * 128 + j * 16
                dst_v[rr, pl.ds(j * 16, 16)] = (
                    (lax.iota(jnp.int32, 16) + i0) * H + h)
            return carry2

        lax.fori_loop(0, SR, mkidx, 0)
        for q in range(NQ):
            cps = []
            for j in range(QR):
                g = q * QR + j
                cps.append(pltpu.async_copy(
                    o_s3.at[inst].at[inv_v.at[g]],
                    obuf.at[pl.ds(j * 128, 128)], sem))
                cps.append(pltpu.async_copy(
                    lse_s3.at[inst].at[inv_v.at[g]],
                    lbuf.at[pl.ds(j * 128, 128)], sem))
            for cp in cps:
                cp.wait()
            cps = []
            for j in range(QR):
                g = q * QR + j
                cps.append(pltpu.async_copy(
                    obuf.at[pl.ds(j * 128, 128)],
                    o_u3.at[rb].at[dst_v.at[g]], sem))
            cps.append(pltpu.async_copy(
                lbuf, lse_u2.at[pl.ds(base_l + q * QR * 128, QR * 128)],
                sem))
            for cp in cps:
                cp.wait()
        return carry

    lax.fori_loop(0, IPW, inst_body, 0)


def k5_unsort_scatter(inv2, o_s, lse_s):
    f = pl.kernel(
        _k5_body,
        mesh=_sc_mesh(),
        compiler_params=pltpu.CompilerParams(use_tc_tiling_on_sc=False),
        out_type=[
            jax.ShapeDtypeStruct((NH * B, S * H, DH), jnp.bfloat16),
            jax.ShapeDtypeStruct((NH * B * H * S, 16), jnp.float32),
        ],
        scratch_types=[
            pltpu.VMEM((SR, 128), jnp.int32),
            pltpu.VMEM((SR, 128), jnp.int32),
            pltpu.VMEM((QR * 128, DH), jnp.bfloat16),
            pltpu.VMEM((QR * 128, 16), jnp.float32),
            pltpu.SemaphoreType.DMA,
        ],
    )
    return f(inv2, o_s, lse_s)


@jax.jit
def kernel(src, Wqk, Wv, Wo, rot, W1, b1, W2, b2):
    qk, v, bkt = k1_proj_hash(src, Wqk, Wv, rot)
    inv = k2_inv(bkt)
    inv2 = inv.transpose(0, 2, 1).reshape(NINST, SR, 128)
    qk_s, v_s = k3_sort_gather(inv2, qk, v)
    o_s, lse_s = k4_attention(qk_s, v_s)
    o_u_tab, lse_u_tab = k5_unsort_scatter(inv2, o_s, lse_s)
    o_u = o_u_tab.reshape(NH, B, S, D)
    lse_u = lse_u_tab.reshape(NH, B, H, S, 16)
    attn_p = k6_combine_wo(o_u, lse_u, Wo)
    return k7_ffn(attn_p, W1, b1, W2, b2)


# speedup vs baseline: 1.2993x; 1.2993x over previous
import jax
import jax.numpy as jnp
import numpy as np
from jax import lax
from jax.experimental import pallas as pl
from jax.experimental.pallas import tpu as pltpu

B, S, D, H = 4, 4096, 1024, 16
DH = D // H
BUCKET = 64
NH = 4
NCH = S // BUCKET
DFF = 4096
NINST = B * H * NH
SCALE = 1.0 / np.sqrt(DH)

BM = 512


def _k1_body(src_ref, wqk_ref, wv_ref, rot_ref, qk_ref, v_ref, bkt_ref):
    x = src_ref[0]
    qk = jnp.dot(x, wqk_ref[...], preferred_element_type=jnp.float32)
    v = jnp.dot(x, wv_ref[...], preferred_element_type=jnp.float32)
    qk_ref[0] = qk.astype(jnp.bfloat16)
    v_ref[0] = v.astype(jnp.bfloat16)
    rot2 = rot_ref[...]
    iota = lax.broadcasted_iota(jnp.int32, (BM, 2 * 32), 1)
    parts = []
    for h in range(H):
        qh = qk[:, h * DH:(h + 1) * DH]
        n = jnp.sqrt(jnp.sum(qh * qh, axis=1, keepdims=True))
        qn = qh / (n + 1e-6)
        rh = jnp.dot(qn, rot2, preferred_element_type=jnp.float32)
        for r in range(NH):
            vals = rh[:, r * 32:(r + 1) * 32]
            cat = jnp.concatenate([vals, -vals], axis=1)
            m = jnp.max(cat, axis=1, keepdims=True)
            idx = jnp.min(jnp.where(cat >= m, iota, 2 * 32), axis=1,
                          keepdims=True)
            parts.append(idx)
    bkt_ref[0] = jnp.concatenate(parts, axis=1)


def k1_proj_hash(src, Wqk, Wv, rot):
    rot2 = rot.reshape(DH, NH * 32)
    return pl.pallas_call(
        _k1_body,
        grid=(B, S // BM),
        in_specs=[
            pl.BlockSpec((1, BM, D), lambda b, s: (b, s, 0)),
            pl.BlockSpec((D, D), lambda b, s: (0, 0)),
            pl.BlockSpec((D, D), lambda b, s: (0, 0)),
            pl.BlockSpec((DH, NH * 32), lambda b, s: (0, 0)),
        ],
        out_specs=[
            pl.BlockSpec((1, BM, D), lambda b, s: (b, s, 0)),
            pl.BlockSpec((1, BM, D), lambda b, s: (b, s, 0)),
            pl.BlockSpec((1, BM, H * NH), lambda b, s: (b, s, 0)),
        ],
        out_shape=[
            jax.ShapeDtypeStruct((B, S, D), jnp.bfloat16),
            jax.ShapeDtypeStruct((B, S, D), jnp.bfloat16),
            jax.ShapeDtypeStruct((B, S, H * NH), jnp.int32),
        ],
    )(src, Wqk, Wv, rot2)


TS = 128
NT = S // TS


def _k2_body(bkt_ref, inv_ref):
    bkt = bkt_ref[0]
    totals = []
    for beta in range(BUCKET):
        totals.append(jnp.sum((bkt == beta).astype(jnp.float32), axis=0,
                              keepdims=True))
    offs = []
    run = jnp.zeros((1, H * NH), jnp.float32)
    for beta in range(BUCKET):
        offs.append(run)
        run = run + totals[beta]
    ii = lax.broadcasted_iota(jnp.int32, (TS, TS), 0)
    jj = lax.broadcasted_iota(jnp.int32, (TS, TS), 1)
    T = (jj <= ii).astype(jnp.float32)

    def tile_step(t, carrys):
        blk = bkt_ref[0, pl.ds(t * TS, TS), :]
        acc = jnp.zeros((TS, H * NH), jnp.float32)
        new_carrys = []
        for beta in range(BUCKET):
            I = (blk == beta).astype(jnp.float32)
            incl = jnp.dot(T, I, preferred_element_type=jnp.float32)
            acc = acc + I * (incl - 1.0 + carrys[beta] + offs[beta])
            new_carrys.append(carrys[beta] + incl[TS - 1:TS, :])
        inv_ref[0, pl.ds(t * TS, TS), :] = acc.astype(jnp.int32)
        return tuple(new_carrys)

    lax.fori_loop(0, NT, tile_step,
                  tuple(jnp.zeros((1, H * NH), jnp.float32)
                        for _ in range(BUCKET)))


def k2_inv(bkt):
    return pl.pallas_call(
        _k2_body,
        grid=(B,),
        in_specs=[pl.BlockSpec((1, S, H * NH), lambda b: (b, 0, 0))],
        out_specs=pl.BlockSpec((1, S, H * NH), lambda b: (b, 0, 0)),
        out_shape=jax.ShapeDtypeStruct((B, S, H * NH), jnp.int32),
    )(bkt)


CG = 4
QB = CG * BUCKET
KB = QB + BUCKET
NB4 = NCH // CG


def _k4_body(qk_ref, v_ref, o_ref, lse_ref, kn_ref):
    qk = qk_ref[0].astype(jnp.float32)
    n = jnp.sqrt(jnp.sum(qk * qk, axis=1, keepdims=True))
    kn_ref[...] = (qk * (1.0 / (n + 1e-6))).astype(jnp.bfloat16)
    ri = lax.broadcasted_iota(jnp.int32, (QB, KB), 0)
    ci = lax.broadcasted_iota(jnp.int32, (QB, KB), 1)
    band = (ci // BUCKET == ri // BUCKET) | (ci // BUCKET == ri // BUCKET + 1)
    dead = (~band) | (ci == ri + BUCKET)
    maskmat = jnp.where(dead, -1e5, 0.0).astype(jnp.float32)

    for cb in range(NB4):
        p0 = ((cb * CG - 1) % NCH) * BUCKET
        qc = qk_ref[0, pl.ds(cb * QB, QB), :]
        kcat = jnp.concatenate(
            [kn_ref[pl.ds(p0, BUCKET), :],
             kn_ref[pl.ds(cb * QB, QB), :]], axis=0)
        vcat = jnp.concatenate(
            [v_ref[0, pl.ds(p0, BUCKET), :],
             v_ref[0, pl.ds(cb * QB, QB), :]], axis=0)
        dots = lax.dot_general(qc, kcat, (((1,), (1,)), ((), ())),
                               preferred_element_type=jnp.float32)
        dots = dots * SCALE + maskmat
        p_ = jnp.exp(dots)
        s = jnp.sum(p_, axis=1, keepdims=True)
        o = jnp.dot(p_.astype(jnp.bfloat16), vcat,
                    preferred_element_type=jnp.float32) * (1.0 / s)
        o_ref[0, pl.ds(cb * QB, QB), :] = o.astype(jnp.bfloat16)
        lse_ref[0, pl.ds(cb * QB, QB), :] = jnp.broadcast_to(
            jnp.log(s), (QB, 16))


def k4_attention(qk_s, v_s):
    return pl.pallas_call(
        _k4_body,
        grid=(NINST,),
        in_specs=[
            pl.BlockSpec((1, S, DH), lambda i: (i, 0, 0)),
            pl.BlockSpec((1, S, DH), lambda i: (i, 0, 0)),
        ],
        out_specs=[
            pl.BlockSpec((1, S, DH), lambda i: (i, 0, 0)),
            pl.BlockSpec((1, S, 16), lambda i: (i, 0, 0)),
        ],
        out_shape=[
            jax.ShapeDtypeStruct((NINST, S, DH), jnp.bfloat16),
            jax.ShapeDtypeStruct((NINST, S, 16), jnp.float32),
        ],
        scratch_shapes=[pltpu.VMEM((S, DH), jnp.bfloat16)],
    )(qk_s, v_s)


BMC = 128


def _k6_body(o_ref, lse_ref, wo_ref, out_ref):
    ls = [lse_ref[r, 0] for r in range(NH)]
    m = ls[0]
    for r in range(1, NH):
        m = jnp.maximum(m, ls[r])
    es = [jnp.exp(l - m) for l in ls]
    ssum = es[0]
    for r in range(1, NH):
        ssum = ssum + es[r]
    bmat = jnp.full((16, DH), 1.0 / 16.0, jnp.float32)
    parts = []
    for h in range(H):
        acc = jnp.zeros((BMC, DH), jnp.float32)
        for r in range(NH):
            w = jnp.dot(es[r][h] * (1.0 / ssum[h]), bmat,
                        preferred_element_type=jnp.float32)
            acc = acc + o_ref[r, 0, :, h * DH:(h + 1) * DH].astype(
                jnp.float32) * w
        parts.append(acc)
    attn = jnp.concatenate(parts, axis=1)
    out_ref[0] = jnp.dot(attn, wo_ref[...], preferred_element_type=jnp.float32)


def k6_combine_wo(o_u, lse_u, Wo):
    return pl.pallas_call(
        _k6_body,
        grid=(B, S // BMC),
        in_specs=[
            pl.BlockSpec((NH, 1, BMC, D), lambda b, s: (0, b, s, 0)),
            pl.BlockSpec((NH, 1, H, BMC, 16), lambda b, s: (0, b, 0, s, 0)),
            pl.BlockSpec((D, D), lambda b, s: (0, 0)),
        ],
        out_specs=pl.BlockSpec((1, BMC, D), lambda b, s: (b, s, 0)),
        out_shape=jax.ShapeDtypeStruct((B, S, D), jnp.float32),
    )(o_u, lse_u, Wo)


FT = 1024


def _k7_body(x_ref, w1_ref, b1_ref, w2_ref, b2_ref, o_ref):
    t = pl.program_id(2)
    h = jnp.dot(x_ref[0], w1_ref[...], preferred_element_type=jnp.float32)
    h = jnp.maximum(h + b1_ref[...], 0.0)
    part = jnp.dot(h, w2_ref[...], preferred_element_type=jnp.float32)

    @pl.when(t == 0)
    def _():
        o_ref[0] = part + b2_ref[...]

    @pl.when(t != 0)
    def _():
        o_ref[0] = o_ref[0] + part


def k7_ffn(x, w1, b1, w2, b2):
    return pl.pallas_call(
        _k7_body,
        grid=(B, S // BM, DFF // FT),
        in_specs=[
            pl.BlockSpec((1, BM, D), lambda b, s, t: (b, s, 0)),
            pl.BlockSpec((D, FT), lambda b, s, t: (0, t)),
            pl.BlockSpec((1, FT), lambda b, s, t: (0, t)),
            pl.BlockSpec((FT, D), lambda b, s, t: (t, 0)),
            pl.BlockSpec((1, D), lambda b, s, t: (0, 0)),
        ],
        out_specs=pl.BlockSpec((1, BM, D), lambda b, s, t: (b, s, 0)),
        out_shape=jax.ShapeDtypeStruct((B, S, D), jnp.float32),
    )(x, w1, b1.reshape(1, DFF), w2, b2.reshape(1, D))

from jax.experimental.pallas import tpu_sc as plsc
import functools

NW = 32
IPW = NINST // NW
SR = S // 128
QR = 4
NQ = SR // QR

def _sc_mesh():
    return plsc.VectorSubcoreMesh(core_axis_name="c", subcore_axis_name="s")


def _k3_body(inv_hbm, qk4, v4, qk_s4, v_s4, inv_v, src_v, qbuf, vbuf, sem):
    wid = lax.axis_index("s") * 2 + lax.axis_index("c")

    def inst_body(k, carry):
        inst = wid * IPW + k
        b = inst // (H * NH)
        c = inst % (H * NH)
        h = c // NH
        pltpu.sync_copy(inv_hbm.at[inst], inv_v)

        def mkidx(rr, carry2):
            for j in range(8):
                i0 = rr * 128 + j * 16
                src_v[rr, pl.ds(j * 16, 16)] = (
                    (lax.iota(jnp.int32, 16) + i0) * H + h)
            return carry2

        lax.fori_loop(0, SR, mkidx, 0)
        for q in range(NQ):
            cps = []
            for j in range(QR):
                g = q * QR + j
                cps.append(pltpu.async_copy(
                    qk4.at[b].at[src_v.at[g]],
                    qbuf.at[pl.ds(j * 128, 128)], sem))
                cps.append(pltpu.async_copy(
                    v4.at[b].at[src_v.at[g]],
                    vbuf.at[pl.ds(j * 128, 128)], sem))
            for cp in cps:
                cp.wait()
            cps = []
            for j in range(QR):
                g = q * QR + j
                cps.append(pltpu.async_copy(
                    qbuf.at[pl.ds(j * 128, 128)],
                    qk_s4.at[inst].at[inv_v.at[g]], sem))
                cps.append(pltpu.async_copy(
                    vbuf.at[pl.ds(j * 128, 128)],
                    v_s4.at[inst].at[inv_v.at[g]], sem))
            for cp in cps:
                cp.wait()
        return carry

    lax.fori_loop(0, IPW, inst_body, 0)


def k3_sort_gather(inv2, qk, v):
    qk4 = qk.reshape(B, S * H, DH)
    v4 = v.reshape(B, S * H, DH)
    f = pl.kernel(
        _k3_body,
        mesh=_sc_mesh(),
        compiler_params=pltpu.CompilerParams(use_tc_tiling_on_sc=False),
        out_type=[
            jax.ShapeDtypeStruct((NINST, S, DH), jnp.bfloat16),
            jax.ShapeDtypeStruct((NINST, S, DH), jnp.bfloat16),
        ],
        scratch_types=[
            pltpu.VMEM((SR, 128), jnp.int32),
            pltpu.VMEM((SR, 128), jnp.int32),
            pltpu.VMEM((QR * 128, DH), jnp.bfloat16),
            pltpu.VMEM((QR * 128, DH), jnp.bfloat16),
            pltpu.SemaphoreType.DMA,
        ],
    )
    return f(inv2, qk4, v4)


def _k5_body(inv_hbm, o_s3, lse_s3, o_u3, lse_u2,
             inv_v, dst_v, obuf, lbuf, sem):
    wid = lax.axis_index("s") * 2 + lax.axis_index("c")

    def inst_body(k, carry):
        inst = wid * IPW + k
        b = inst // (H * NH)
        c = inst % (H * NH)
        h = c // NH
        r = c % NH
        rb = r * B + b
        base_l = ((rb * H) + h) * S
        pltpu.sync_copy(inv_hbm.at[inst], inv_v)

        def mkidx(rr, carry2):
            for j in range(8):
                i0 = rr * 128 + j * 16
                dst_v[rr, pl.ds(j * 16, 16)] = (
                    (lax.iota(jnp.int32, 16) + i0) * H + h)
            return carry2

        lax.fori_loop(0, SR, mkidx, 0)
        for q in range(NQ):
            cps = []
            for j in range(QR):
                g = q * QR + j
                cps.append(pltpu.async_copy(
                    o_s3.at[inst].at[inv_v.at[g]],
                    obuf.at[pl.ds(j * 128, 128)], sem))
                cps.append(pltpu.async_copy(
                    lse_s3.at[inst].at[inv_v.at[g]],
                    lbuf.at[pl.ds(j * 128, 128)], sem))
            for cp in cps:
                cp.wait()
            cps = []
            for j in range(QR):
                g = q * QR + j
                cps.append(pltpu.async_copy(
                    obuf.at[pl.ds(j * 128, 128)],
                    o_u3.at[rb].at[dst_v.at[g]], sem))
            cps.append(pltpu.async_copy(
                lbuf, lse_u2.at[pl.ds(base_l + q * QR * 128, QR * 128)],
                sem))
            for cp in cps:
                cp.wait()
        return carry

    lax.fori_loop(0, IPW, inst_body, 0)


def k5_unsort_scatter(inv2, o_s, lse_s):
    f = pl.kernel(
        _k5_body,
        mesh=_sc_mesh(),
        compiler_params=pltpu.CompilerParams(use_tc_tiling_on_sc=False),
        out_type=[
            jax.ShapeDtypeStruct((NH * B, S * H, DH), jnp.bfloat16),
            jax.ShapeDtypeStruct((NH * B * H * S, 16), jnp.float32),
        ],
        scratch_types=[
            pltpu.VMEM((SR, 128), jnp.int32),
            pltpu.VMEM((SR, 128), jnp.int32),
            pltpu.VMEM((QR * 128, DH), jnp.bfloat16),
            pltpu.VMEM((QR * 128, 16), jnp.float32),
            pltpu.SemaphoreType.DMA,
        ],
    )
    return f(inv2, o_s, lse_s)


@jax.jit
def kernel(src, Wqk, Wv, Wo, rot, W1, b1, W2, b2):
    qk, v, bkt = k1_proj_hash(src, Wqk, Wv, rot)
    inv = k2_inv(bkt)
    inv2 = inv.transpose(0, 2, 1).reshape(NINST, SR, 128)
    qk_s, v_s = k3_sort_gather(inv2, qk, v)
    o_s, lse_s = k4_attention(qk_s, v_s)
    o_u_tab, lse_u_tab = k5_unsort_scatter(inv2, o_s, lse_s)
    o_u = o_u_tab.reshape(NH, B, S, D)
    lse_u = lse_u_tab.reshape(NH, B, H, S, 16)
    attn_p = k6_combine_wo(o_u, lse_u, Wo)
    return k7_ffn(attn_p, W1, b1, W2, b2)
